# baseline (speedup 1.0000x reference)
import jax
import jax.numpy as jnp
from jax import lax
from jax.experimental import pallas as pl
from jax.experimental.pallas import tpu as pltpu
from jax.experimental.pallas import tpu_sc as plsc

SEQ = 512
DIAG = 2
NROWS = SEQ - DIAG
TRI = NROWS * (NROWS + 1) // 2
NMAT = 128
NC, NS = 2, 16
NW = NC * NS
MPW = NMAT // NW
R = 16
NBLK = 32
NRING = 4
NOUT = 4


def _off(i: int) -> int:
    return NROWS * i - i * (i - 1) // 2


OFF = [_off(R * b) for b in range(NBLK)]
FLUSH_LAST = 104
SBF = [OFF[b + 1] - OFF[b] for b in range(NBLK - 1)] + [FLUSH_LAST]
MAT_FLUSHED = OFF[NBLK - 1] + FLUSH_LAST
MATW = SEQ * SEQ
BLOCK_WORDS = R * SEQ
OUT_WORDS = NMAT * TRI
OUT_STAGE = SBF[0] + 40


def _body(in1d, in2d, out_hbm, *scratch):
    ibufa = scratch[:NRING]
    ibufb = scratch[NRING:2 * NRING]
    obufs = scratch[2 * NRING:2 * NRING + NOUT]
    isema = scratch[2 * NRING + NOUT:3 * NRING + NOUT]
    isemb = scratch[3 * NRING + NOUT:4 * NRING + NOUT]
    osems = scratch[4 * NRING + NOUT:]
    c = lax.axis_index("c")
    s = lax.axis_index("s")
    w = s * NC + c
    par = w % 2
    m0 = w * MPW
    span = m0 * TRI - 4 * par
    c0 = 4 * par

    @pl.when(par == 1)
    def _head_patch():
        pbase = (m0 - 1) * MATW
        for n, chunk in enumerate((260088, 260600, 261112)):
            pltpu.sync_copy(in1d.at[pl.ds(pl.multiple_of(pbase + chunk, 8),
                                          8)],
                            ibufa[0].at[pl.ds(8 * n, 8)])
        v0 = ibufa[0][pl.ds(0, 16)]
        v1 = ibufa[0][pl.ds(8, 16)]
        v2 = ibufa[0][pl.ds(16, 16)]
        lane = lax.iota(jnp.int32, 16)
        patch = jnp.where(lane == 0, v0[7],
                          jnp.where(lane == 1, v1[6],
                                    jnp.where(lane == 2, v1[7], v2[7])))
        obufs[NOUT - 1][pl.ds(FLUSH_LAST, 16)] = patch

    def in_issue(mrow, b):
        if b < 8:
            src = in1d.at[pl.ds(pl.multiple_of(mrow * SEQ + SEQ * R * b, 8),
                                BLOCK_WORDS)]
            pltpu.async_copy(src, ibufa[b % NRING].at[pl.ds(0, BLOCK_WORDS)],
                             isema[b % NRING])
        else:
            q = b // 8
            for cc in range(q, 4):
                src = in2d.at[pl.ds(pl.multiple_of(mrow + R * b, 8), R),
                              pl.ds(128 * cc, 128)]
                pltpu.async_copy(
                    src,
                    ibufb[b % NRING].at[pl.ds((cc - q) * 16, 16),
                                        pl.ds(0, 128)],
                    isemb[b % NRING])

    def in_drain(b):
        if b < 8:
            pltpu.make_async_copy(
                in1d.at[pl.ds(0, BLOCK_WORDS)],
                ibufa[b % NRING].at[pl.ds(0, BLOCK_WORDS)],
                isema[b % NRING]).wait()
        else:
            n = (4 - b // 8) * 16
            pltpu.make_async_copy(
                in2d.at[pl.ds(0, n), pl.ds(0, 128)],
                ibufb[b % NRING].at[pl.ds(0, n), pl.ds(0, 128)],
                isemb[b % NRING]).wait()

    def compact(b, cpos, obuf):
        r0 = R * b
        q = b // 8

        if b < 8:
            ibuf = ibufa[b % NRING]

            def row_body(j, _):
                i = r0 + j
                length = NROWS - i
                nk = (length + 15) // 16
                dst0 = cpos + (NROWS * i - i * (i - 1) // 2) - _off(r0)
                src0 = SEQ * j + i + DIAG

                @plsc.parallel_loop(0, nk, 1, unroll=4)
                def _k(k):
                    obuf[pl.ds(dst0 + 16 * k, 16)] = (
                        ibuf[pl.ds(src0 + 16 * k, 16)])

                return 0
        else:
            ibuf = ibufb[b % NRING]

            def row_body(j, _):
                i = r0 + j
                base_dst = cpos + (NROWS * i - i * (i - 1) // 2) - _off(r0)
                for cc in range(q, 4):
                    col0 = jnp.minimum(
                        jnp.maximum(i + DIAG - 128 * cc, 0), 128)
                    nk = (128 - col0 + 15) // 16
                    dst0 = base_dst + jnp.maximum(128 * cc - (i + DIAG), 0)
                    srow = (cc - q) * 16 + j

                    @plsc.parallel_loop(0, nk, 1, unroll=4)
                    def _k(k, dst0=dst0, col0=col0, srow=srow):
                        obuf[pl.ds(dst0 + 16 * k, 16)] = (
                            ibuf[srow, pl.ds(col0 + 16 * k, 16)])

                return 0

        lax.fori_loop(0, R, row_body, 0)

    def mat_body(mat, _):
        mrow = (m0 + mat) * SEQ
        cpos = c0 + mat
        obase = span + mat * MAT_FLUSHED
        h_out = [None] * NBLK

        @pl.when(mat == 0)
        def _prologue():
            for p in range(NRING - 1):
                in_issue(mrow, p)

        for b in range(NBLK):
            cur = b % NOUT
            nb = b + NRING - 1
            if nb < NBLK:
                in_issue(mrow, nb)
            else:
                @pl.when(mat + 1 < MPW)
                def _prefetch_next(nb=nb):
                    in_issue(mrow + SEQ, nb - NBLK)
            in_drain(b)
            if b >= NOUT:
                h_out[b - NOUT].wait()
            prev_flush = FLUSH_LAST if b == 0 else SBF[b - 1]
            obufs[cur][pl.ds(0, 16)] = (
                obufs[(b - 1) % NOUT][pl.ds(prev_flush, 16)])
            compact(b, cpos, obufs[cur])
            h_out[b] = pltpu.async_copy(
                obufs[cur].at[pl.ds(0, SBF[b])],
                out_hbm.at[pl.ds(pl.multiple_of(obase + OFF[b], 8), SBF[b])],
                osems[cur])
        for b in range(NBLK - NOUT, NBLK):
            h_out[b].wait()
        return 0

    lax.fori_loop(0, MPW, mat_body, 0)

    @pl.when(par == 1)
    def _tail_flush():
        pltpu.sync_copy(
            obufs[(NBLK - 1) % NOUT].at[pl.ds(FLUSH_LAST, 8)],
            out_hbm.at[pl.ds(pl.multiple_of(span + MPW * MAT_FLUSHED, 8), 8)])


@jax.jit
def _upper_tri(in1d, in2d):
    mesh = plsc.VectorSubcoreMesh(core_axis_name="c", subcore_axis_name="s",
                                  num_cores=NC, num_subcores=NS)
    return pl.kernel(
        _body,
        out_type=jax.ShapeDtypeStruct((OUT_WORDS,), jnp.float32),
        mesh=mesh,
        scratch_types=(
            [pltpu.VMEM((BLOCK_WORDS + 32,), jnp.float32)] * NRING
            + [pltpu.VMEM((56, 128), jnp.float32)] * NRING
            + [pltpu.VMEM((OUT_STAGE,), jnp.float32)] * NOUT
            + [pltpu.SemaphoreType.DMA] * (2 * NRING + NOUT)
        ),
    )(in1d, in2d)


def kernel(inputs):
    batch, chan, seq, _ = inputs.shape
    in1d = inputs.reshape(batch * chan * seq * seq)
    in2d = inputs.reshape(batch * chan * seq, seq)
    out = _upper_tri(in1d, in2d)
    return out.reshape(batch, chan, TRI)


# speedup vs baseline: 1.1354x; 1.1354x over previous
import jax
import jax.numpy as jnp
from jax import lax
from jax.experimental import pallas as pl
from jax.experimental.pallas import tpu as pltpu
from jax.experimental.pallas import tpu_sc as plsc

SEQ = 512
DIAG = 2
NROWS = SEQ - DIAG
TRI = NROWS * (NROWS + 1) // 2
NMAT = 128
NC, NS = 2, 16
NW = NC * NS
MPW = NMAT // NW
R = 16
NBLK = 32
NRING = 8
NOUT = 4


def _off(i: int) -> int:
    return NROWS * i - i * (i - 1) // 2


OFF = [_off(R * b) for b in range(NBLK)]
FLUSH_LAST = 104
SBF = [OFF[b + 1] - OFF[b] for b in range(NBLK - 1)] + [FLUSH_LAST]
MAT_FLUSHED = OFF[NBLK - 1] + FLUSH_LAST
MATW = SEQ * SEQ
BLOCK_WORDS = R * SEQ
OUT_WORDS = NMAT * TRI
OUT_STAGE = SBF[0] + 40


def _body(in_hbm, out_hbm, *scratch):
    ibufs = scratch[:NRING]
    obufs = scratch[NRING:NRING + NOUT]
    isems = scratch[NRING + NOUT:2 * NRING + NOUT]
    osems = scratch[2 * NRING + NOUT:]
    c = lax.axis_index("c")
    s = lax.axis_index("s")
    w = s * NC + c
    par = w % 2
    m0 = w * MPW
    span = m0 * TRI - 4 * par
    c0 = 4 * par

    @pl.when(par == 1)
    def _head_patch():
        prow = m0 * SEQ - SEQ
        for n, row in enumerate((507, 508, 509)):
            pltpu.sync_copy(in_hbm.at[prow + row, pl.ds(504, 8)],
                            ibufs[0].at[0, pl.ds(8 * n, 8)])
        v0 = ibufs[0][0, pl.ds(0, 16)]
        v1 = ibufs[0][0, pl.ds(8, 16)]
        v2 = ibufs[0][0, pl.ds(16, 16)]
        lane = lax.iota(jnp.int32, 16)
        patch = jnp.where(lane == 0, v0[7],
                          jnp.where(lane == 1, v1[6],
                                    jnp.where(lane == 2, v1[7], v2[7])))
        obufs[NOUT - 1][pl.ds(FLUSH_LAST, 16)] = patch

    def in_issue(mrow, b):
        q = b // 8
        for cc in range(q, 4):
            src = in_hbm.at[pl.ds(pl.multiple_of(mrow + R * b, 8), R),
                            pl.ds(128 * cc, 128)]
            pltpu.async_copy(
                src,
                ibufs[b % NRING].at[pl.ds((cc - q) * 16, 16), pl.ds(0, 128)],
                isems[b % NRING])

    def in_drain(b):
        n = (4 - b // 8) * 16
        pltpu.make_async_copy(
            in_hbm.at[pl.ds(0, n), pl.ds(0, 128)],
            ibufs[b % NRING].at[pl.ds(0, n), pl.ds(0, 128)],
            isems[b % NRING]).wait()

    def compact(b, cpos, obuf):
        r0 = R * b
        q = b // 8
        ibuf = ibufs[b % NRING]

        def row_body(j, _):
            i = r0 + j
            base_dst = cpos + (NROWS * i - i * (i - 1) // 2) - _off(r0)
            for cc in range(q, 4):
                col0 = jnp.minimum(jnp.maximum(i + DIAG - 128 * cc, 0), 128)
                nk = (128 - col0 + 15) // 16
                dst0 = base_dst + jnp.maximum(128 * cc - (i + DIAG), 0)
                srow = (cc - q) * 16 + j

                @plsc.parallel_loop(0, nk, 1, unroll=4)
                def _k(k, dst0=dst0, col0=col0, srow=srow):
                    obuf[pl.ds(dst0 + 16 * k, 16)] = (
                        ibuf[srow, pl.ds(col0 + 16 * k, 16)])

            return 0

        lax.fori_loop(0, R, row_body, 0)

    def mat_body(mat, _):
        mrow = (m0 + mat) * SEQ
        cpos = c0 + mat
        obase = span + mat * MAT_FLUSHED
        h_out = [None] * NBLK

        @pl.when(mat == 0)
        def _prologue():
            for p in range(NRING - 1):
                in_issue(mrow, p)

        for b in range(NBLK):
            cur = b % NOUT
            nb = b + NRING - 1
            if nb < NBLK:
                in_issue(mrow, nb)
            else:
                @pl.when(mat + 1 < MPW)
                def _prefetch_next(nb=nb):
                    in_issue(mrow + SEQ, nb - NBLK)
            in_drain(b)
            if b >= NOUT:
                h_out[b - NOUT].wait()
            prev_flush = FLUSH_LAST if b == 0 else SBF[b - 1]
            obufs[cur][pl.ds(0, 16)] = (
                obufs[(b - 1) % NOUT][pl.ds(prev_flush, 16)])
            compact(b, cpos, obufs[cur])
            h_out[b] = pltpu.async_copy(
                obufs[cur].at[pl.ds(0, SBF[b])],
                out_hbm.at[pl.ds(pl.multiple_of(obase + OFF[b], 8), SBF[b])],
                osems[cur])
        for b in range(NBLK - NOUT, NBLK):
            h_out[b].wait()
        return 0

    lax.fori_loop(0, MPW, mat_body, 0)

    @pl.when(par == 1)
    def _tail_flush():
        pltpu.sync_copy(
            obufs[(NBLK - 1) % NOUT].at[pl.ds(FLUSH_LAST, 8)],
            out_hbm.at[pl.ds(pl.multiple_of(span + MPW * MAT_FLUSHED, 8), 8)])


@jax.jit
def _upper_tri(in2d):
    mesh = plsc.VectorSubcoreMesh(core_axis_name="c", subcore_axis_name="s",
                                  num_cores=NC, num_subcores=NS)
    return pl.kernel(
        _body,
        out_type=jax.ShapeDtypeStruct((OUT_WORDS,), jnp.float32),
        mesh=mesh,
        scratch_types=(
            [pltpu.VMEM((72, 128), jnp.float32)] * NRING
            + [pltpu.VMEM((OUT_STAGE,), jnp.float32)] * NOUT
            + [pltpu.SemaphoreType.DMA] * (NRING + NOUT)
        ),
    )(in2d)


def kernel(inputs):
    batch, chan, seq, _ = inputs.shape
    in2d = inputs.reshape(batch * chan * seq, seq)
    out = _upper_tri(in2d)
    return out.reshape(batch, chan, TRI)
